# baseline (speedup 1.0000x reference)
import functools

import jax
import jax.numpy as jnp
from jax import lax
from jax.experimental import pallas as pl
from jax.experimental.pallas import tpu as pltpu
from jax.experimental.pallas import tpu_sc as plsc

N = 10000
E = 320000
DF = 128
NP = 10240
NCHUNK = 128
NWORKERS = 32
TOTAL_CHUNKS = 2560
EP = TOTAL_CHUNKS * NCHUNK
KPT = TOTAL_CHUNKS // NWORKERS
GC = 40
GROUPS = KPT // GC
RPT = NP // 16
RB = 1000
_BN_SCALE = 0.9999950000374996


DR = 128


def _sc_degree(dstp, mesh):
    @functools.partial(
        pl.kernel,
        out_type=jax.ShapeDtypeStruct((2, DR, 128), jnp.float32),
        mesh=mesh,
        compiler_params=pltpu.CompilerParams(needs_layout_passes=False),
        scratch_types=[
            pltpu.VMEM((KPT, NCHUNK), jnp.int32),
            pltpu.VMEM((DR, 128), jnp.float32),
            pltpu.VMEM((DR // 16, 128), jnp.float32),
            pltpu.VMEM((1, 128), jnp.int32),
            pltpu.VMEM_SHARED((DR, 128), jnp.float32),
        ],
    )
    def k(dstp_hbm, out_hbm, dbuf, hist, zbuf, idxrow, dacc):
        c = lax.axis_index("c")
        s = lax.axis_index("s")
        wid = s * 2 + c
        zrows = DR // 16
        lane16 = jax.lax.iota(jnp.int32, 16)

        pltpu.sync_copy(dstp_hbm.at[pl.ds(wid * KPT, KPT)], dbuf)

        def fill_zero(i, carry):
            r = i // 8
            k16 = i % 8
            hist[r, pl.ds(k16 * 16, 16)] = jnp.zeros((16,), jnp.float32)
            return carry

        lax.fori_loop(0, DR * 8, fill_zero, 0)

        def fill_zbuf(i, carry):
            zbuf[i // 8, pl.ds((i % 8) * 16, 16)] = jnp.zeros((16,), jnp.float32)
            return carry

        lax.fori_loop(0, zrows * 8, fill_zbuf, 0)

        for k16 in range(8):
            idxrow[0, pl.ds(k16 * 16, 16)] = lane16 + (16 * k16)

        pltpu.sync_copy(zbuf, dacc.at[pl.ds(s * zrows, zrows)])

        def count(i, carry):
            v = dbuf[i // 8, pl.ds((i % 8) * 16, 16)]
            cnt, is_last = plsc.scan_count(v)
            plsc.addupdate_scatter(
                hist,
                [lax.shift_right_logical(v, 7), lax.bitwise_and(v, 127)],
                cnt.astype(jnp.float32),
                mask=is_last,
            )
            return carry

        lax.fori_loop(0, KPT * 8, count, 0)
        plsc.subcore_barrier()
        pltpu.sync_copy(hist, dacc.at[idxrow.at[0]], add=True)
        plsc.subcore_barrier()
        pltpu.sync_copy(dacc.at[pl.ds(s * zrows, zrows)],
                        out_hbm.at[c, pl.ds(s * zrows, zrows)])

    return k(dstp)


def _sc_scatter(u, srcp, dstp, z128, mesh):
    @functools.partial(
        pl.kernel,
        out_type=jax.ShapeDtypeStruct((2, NP, DF), jnp.float32),
        mesh=mesh,
        scratch_types=[
            pltpu.VMEM((GC, NCHUNK), jnp.int32),
            pltpu.VMEM((GC, NCHUNK), jnp.int32),
            pltpu.VMEM((NCHUNK, DF), jnp.float32),
            pltpu.VMEM((NCHUNK, DF), jnp.float32),
            pltpu.VMEM_SHARED((NP, DF), jnp.float32),
            pltpu.SemaphoreType.DMA,
            pltpu.SemaphoreType.DMA,
            pltpu.SemaphoreType.DMA,
            pltpu.SemaphoreType.DMA,
            pltpu.SemaphoreType.DMA,
        ],
    )
    def k(u_hbm, srcp_hbm, dstp_hbm, z_hbm, out_hbm,
          sbuf, dbuf, rows0, rows1, acc, sem0, sem1, sems0, sems1, semz):
        c = lax.axis_index("c")
        s = lax.axis_index("s")
        wid = s * 2 + c

        def load_idx(gi):
            base = wid * KPT + gi * GC
            pltpu.sync_copy(srcp_hbm.at[pl.ds(base, GC)], sbuf)
            pltpu.sync_copy(dstp_hbm.at[pl.ds(base, GC)], dbuf)

        def gather(j, rows, sem):
            pltpu.async_copy(u_hbm.at[sbuf.at[j]], rows, sem)

        def gather_wait(j, rows, sem):
            pltpu.make_async_copy(u_hbm.at[sbuf.at[j]], rows, sem).wait()

        zc = pltpu.async_copy(z_hbm.at[pl.ds(s * RPT, RPT)],
                              acc.at[pl.ds(s * RPT, RPT)], semz)
        load_idx(0)
        gather(0, rows0, sem0)
        gather(1, rows1, sem1)
        zc.wait()
        plsc.subcore_barrier()

        def body(i, carry):
            j = 2 * i
            gather_wait(j, rows0, sem0)
            sc0 = pltpu.async_copy(rows0, acc.at[dbuf.at[j]], sems0, add=True)
            gather_wait(j + 1, rows1, sem1)
            sc1 = pltpu.async_copy(rows1, acc.at[dbuf.at[j + 1]], sems1, add=True)
            sc0.wait()

            @pl.when(j + 2 < GC)
            def _():
                gather(j + 2, rows0, sem0)

            sc1.wait()

            @pl.when(j + 3 < GC)
            def _():
                gather(j + 3, rows1, sem1)

            return carry

        lax.fori_loop(0, GC // 2, body, 0)
        for gi in range(1, GROUPS):
            load_idx(gi)
            gather(0, rows0, sem0)
            gather(1, rows1, sem1)
            lax.fori_loop(0, GC // 2, body, 0)
        plsc.subcore_barrier()
        pltpu.sync_copy(acc.at[pl.ds(s * RPT, RPT)],
                        out_hbm.at[c, pl.ds(s * RPT, RPT)])

    return k(u, srcp, dstp, z128)


def _tc_first_scale(dflat, x, W, Wm):
    def body(d_ref, x_ref, w_ref, wm_ref, dis_ref, u_ref):
        cnt = d_ref[0] + d_ref[1]
        dis = lax.rsqrt(1.0 + cnt)
        dis_ref[...] = dis
        mw = jnp.clip(wm_ref[...], 0.0, 1.0) * w_ref[...]
        u_ref[...] = jnp.dot(x_ref[...], mw,
                             preferred_element_type=jnp.float32) * dis

    return pl.pallas_call(
        body,
        grid=(N // RB,),
        in_specs=[
            pl.BlockSpec((2, RB, 1), lambda i: (0, i, 0)),
            pl.BlockSpec((RB, DF), lambda i: (i, 0)),
            pl.BlockSpec((DF, DF), lambda i: (0, 0)),
            pl.BlockSpec((DF, DF), lambda i: (0, 0)),
        ],
        out_specs=[
            pl.BlockSpec((RB, 1), lambda i: (i, 0)),
            pl.BlockSpec((RB, DF), lambda i: (i, 0)),
        ],
        out_shape=[
            jax.ShapeDtypeStruct((N, 1), jnp.float32),
            jax.ShapeDtypeStruct((N, DF), jnp.float32),
        ],
    )(dflat, x, W, Wm)


def _tc_layer(sacc, u, dis, b, bm, g, be, Wn, Wmn):
    def body(s_ref, u_ref, dis_ref, b_ref, bm_ref, g_ref, be_ref,
             w_ref, wm_ref, un_ref):
        dis_b = dis_ref[...]
        o = (s_ref[0] + s_ref[1] + u_ref[...]) * dis_b \
            + jnp.clip(bm_ref[...], 0.0, 1.0) * b_ref[...]
        a = jnp.maximum(o * _BN_SCALE * g_ref[...] + be_ref[...], 0.0)
        mw = jnp.clip(wm_ref[...], 0.0, 1.0) * w_ref[...]
        un_ref[...] = jnp.dot(a, mw, preferred_element_type=jnp.float32) * dis_b

    full = lambda i: (0, 0)
    return pl.pallas_call(
        body,
        grid=(N // RB,),
        in_specs=[
            pl.BlockSpec((2, RB, DF), lambda i: (0, i, 0)),
            pl.BlockSpec((RB, DF), lambda i: (i, 0)),
            pl.BlockSpec((RB, 1), lambda i: (i, 0)),
            pl.BlockSpec((1, DF), full),
            pl.BlockSpec((1, DF), full),
            pl.BlockSpec((1, DF), full),
            pl.BlockSpec((1, DF), full),
            pl.BlockSpec((DF, DF), full),
            pl.BlockSpec((DF, DF), full),
        ],
        out_specs=pl.BlockSpec((RB, DF), lambda i: (i, 0)),
        out_shape=jax.ShapeDtypeStruct((N, DF), jnp.float32),
    )(sacc, u, dis, b, bm, g, be, Wn, Wmn)


def _tc_final(sacc, u, dis, b, bm):
    def body(s_ref, u_ref, dis_ref, b_ref, bm_ref, out_ref):
        o = (s_ref[0] + s_ref[1] + u_ref[...]) * dis_ref[...] \
            + jnp.clip(bm_ref[...], 0.0, 1.0) * b_ref[...]
        m = jnp.max(o, axis=-1, keepdims=True)
        z = o - m
        lse = jnp.log(jnp.sum(jnp.exp(z), axis=-1, keepdims=True))
        out_ref[...] = z - lse

    full = lambda i: (0, 0)
    return pl.pallas_call(
        body,
        grid=(N // RB,),
        in_specs=[
            pl.BlockSpec((2, RB, DF), lambda i: (0, i, 0)),
            pl.BlockSpec((RB, DF), lambda i: (i, 0)),
            pl.BlockSpec((RB, 1), lambda i: (i, 0)),
            pl.BlockSpec((1, DF), full),
            pl.BlockSpec((1, DF), full),
        ],
        out_specs=pl.BlockSpec((RB, DF), lambda i: (i, 0)),
        out_shape=jax.ShapeDtypeStruct((N, DF), jnp.float32),
    )(sacc, u, dis, b, bm)


def kernel(x, edge_index, W1, Wm1, b1, bm1, g1, be1,
           W2, Wm2, b2, bm2, g2, be2, W3, Wm3, b3, bm3):
    src = edge_index[0]
    dst = edge_index[1]
    pad_i = jnp.arange(EP - E, dtype=jnp.int32)
    psrc = jnp.concatenate([src, pad_i % N])
    pdst = jnp.concatenate([dst, N + pad_i % (NP - N)])
    srcp = psrc.reshape(TOTAL_CHUNKS, NCHUNK)
    dstp = pdst.reshape(TOTAL_CHUNKS, NCHUNK)

    z128 = jnp.zeros((NP, DF), jnp.float32)

    b1r, bm1r = b1.reshape(1, DF), bm1.reshape(1, DF)
    b2r, bm2r = b2.reshape(1, DF), bm2.reshape(1, DF)
    b3r, bm3r = b3.reshape(1, DF), bm3.reshape(1, DF)
    g1r, be1r = g1.reshape(1, DF), be1.reshape(1, DF)
    g2r, be2r = g2.reshape(1, DF), be2.reshape(1, DF)

    mesh = plsc.VectorSubcoreMesh(core_axis_name="c", subcore_axis_name="s")

    dacc = _sc_degree(dstp, mesh)
    dflat = dacc.reshape(2, DR * 128, 1)
    dis, u1 = _tc_first_scale(dflat, x, W1, Wm1)
    s1 = _sc_scatter(u1, srcp, dstp, z128, mesh)
    u2 = _tc_layer(s1, u1, dis, b1r, bm1r, g1r, be1r, W2, Wm2)
    s2 = _sc_scatter(u2, srcp, dstp, z128, mesh)
    u3 = _tc_layer(s2, u2, dis, b2r, bm2r, g2r, be2r, W3, Wm3)
    s3 = _sc_scatter(u3, srcp, dstp, z128, mesh)
    return _tc_final(s3, u3, dis, b3r, bm3r)


# speedup vs baseline: 1.2478x; 1.2478x over previous
import functools

import jax
import jax.numpy as jnp
from jax import lax
from jax.experimental import pallas as pl
from jax.experimental.pallas import tpu as pltpu
from jax.experimental.pallas import tpu_sc as plsc

N = 10000
E = 320000
DF = 128
NP = 10240
NCHUNK = 128
NWORKERS = 32
TOTAL_CHUNKS = 2560
EP = TOTAL_CHUNKS * NCHUNK
KPT = TOTAL_CHUNKS // NWORKERS
GC = 40
GROUPS = KPT // GC
RPT = NP // 16
RB = 1000
_BN_SCALE = 0.9999950000374996


DR = 128


def _sc_degree(dstp, mesh):
    @functools.partial(
        pl.kernel,
        out_type=jax.ShapeDtypeStruct((2, DR, 128), jnp.float32),
        mesh=mesh,
        compiler_params=pltpu.CompilerParams(needs_layout_passes=False),
        scratch_types=[
            pltpu.VMEM((KPT, NCHUNK), jnp.int32),
            pltpu.VMEM((DR, 128), jnp.float32),
            pltpu.VMEM((DR // 16, 128), jnp.float32),
            pltpu.VMEM((1, 128), jnp.int32),
            pltpu.VMEM_SHARED((DR, 128), jnp.float32),
        ],
    )
    def k(dstp_hbm, out_hbm, dbuf, hist, zbuf, idxrow, dacc):
        c = lax.axis_index("c")
        s = lax.axis_index("s")
        wid = s * 2 + c
        zrows = DR // 16
        lane16 = jax.lax.iota(jnp.int32, 16)

        pltpu.sync_copy(dstp_hbm.at[pl.ds(wid * KPT, KPT)], dbuf)

        def fill_zero(i, carry):
            r = i // 8
            k16 = i % 8
            hist[r, pl.ds(k16 * 16, 16)] = jnp.zeros((16,), jnp.float32)
            return carry

        lax.fori_loop(0, DR * 8, fill_zero, 0)

        def fill_zbuf(i, carry):
            zbuf[i // 8, pl.ds((i % 8) * 16, 16)] = jnp.zeros((16,), jnp.float32)
            return carry

        lax.fori_loop(0, zrows * 8, fill_zbuf, 0)

        for k16 in range(8):
            idxrow[0, pl.ds(k16 * 16, 16)] = lane16 + (16 * k16)

        pltpu.sync_copy(zbuf, dacc.at[pl.ds(s * zrows, zrows)])

        def count(i, carry):
            v = dbuf[i // 8, pl.ds((i % 8) * 16, 16)]
            cnt, is_last = plsc.scan_count(v)
            plsc.addupdate_scatter(
                hist,
                [lax.shift_right_logical(v, 7), lax.bitwise_and(v, 127)],
                cnt.astype(jnp.float32),
                mask=is_last,
            )
            return carry

        lax.fori_loop(0, KPT * 8, count, 0)
        plsc.subcore_barrier()
        pltpu.sync_copy(hist, dacc.at[idxrow.at[0]], add=True)
        plsc.subcore_barrier()
        pltpu.sync_copy(dacc.at[pl.ds(s * zrows, zrows)],
                        out_hbm.at[c, pl.ds(s * zrows, zrows)])

    return k(dstp)


def _sc_scatter(u, srcp, dstp, z128, mesh):
    @functools.partial(
        pl.kernel,
        out_type=jax.ShapeDtypeStruct((2, NP, DF), jnp.float32),
        mesh=mesh,
        scratch_types=[
            pltpu.VMEM((GC, NCHUNK), jnp.int32),
            pltpu.VMEM((GC, NCHUNK), jnp.int32),
            pltpu.VMEM((NCHUNK, DF), jnp.float32),
            pltpu.VMEM((NCHUNK, DF), jnp.float32),
            pltpu.VMEM_SHARED((NP, DF), jnp.float32),
            pltpu.SemaphoreType.DMA,
            pltpu.SemaphoreType.DMA,
            pltpu.SemaphoreType.DMA,
            pltpu.SemaphoreType.DMA,
            pltpu.SemaphoreType.DMA,
        ],
    )
    def k(u_hbm, srcp_hbm, dstp_hbm, z_hbm, out_hbm,
          sbuf, dbuf, rows0, rows1, acc, sem0, sem1, sems0, sems1, semz):
        c = lax.axis_index("c")
        s = lax.axis_index("s")
        wid = s * 2 + c

        def load_idx(gi):
            base = wid * KPT + gi * GC
            pltpu.sync_copy(srcp_hbm.at[pl.ds(base, GC)], sbuf)
            pltpu.sync_copy(dstp_hbm.at[pl.ds(base, GC)], dbuf)

        def gather(j, rows, sem):
            pltpu.async_copy(u_hbm.at[sbuf.at[j]], rows, sem)

        def gather_wait(j, rows, sem):
            pltpu.make_async_copy(u_hbm.at[sbuf.at[j]], rows, sem).wait()

        zc = pltpu.async_copy(z_hbm.at[pl.ds(s * RPT, RPT)],
                              acc.at[pl.ds(s * RPT, RPT)], semz)
        load_idx(0)
        gather(0, rows0, sem0)
        gather(1, rows1, sem1)
        zc.wait()
        plsc.subcore_barrier()

        def body(i, carry):
            j = 2 * i
            gather_wait(j, rows0, sem0)
            pltpu.sync_copy(rows0, acc.at[dbuf.at[j]], add=True)

            @pl.when(j + 2 < GC)
            def _():
                gather(j + 2, rows0, sem0)

            gather_wait(j + 1, rows1, sem1)
            pltpu.sync_copy(rows1, acc.at[dbuf.at[j + 1]], add=True)

            @pl.when(j + 3 < GC)
            def _():
                gather(j + 3, rows1, sem1)

            return carry

        lax.fori_loop(0, GC // 2, body, 0)
        for gi in range(1, GROUPS):
            load_idx(gi)
            gather(0, rows0, sem0)
            gather(1, rows1, sem1)
            lax.fori_loop(0, GC // 2, body, 0)
        plsc.subcore_barrier()
        pltpu.sync_copy(acc.at[pl.ds(s * RPT, RPT)],
                        out_hbm.at[c, pl.ds(s * RPT, RPT)])

    return k(u, srcp, dstp, z128)


def _tc_first_scale(dflat, x, W, Wm):
    def body(d_ref, x_ref, w_ref, wm_ref, dis_ref, u_ref):
        cnt = d_ref[0] + d_ref[1]
        dis = lax.rsqrt(1.0 + cnt)
        dis_ref[...] = dis
        mw = jnp.clip(wm_ref[...], 0.0, 1.0) * w_ref[...]
        u_ref[...] = jnp.dot(x_ref[...], mw,
                             preferred_element_type=jnp.float32) * dis

    return pl.pallas_call(
        body,
        grid=(N // RB,),
        in_specs=[
            pl.BlockSpec((2, RB, 1), lambda i: (0, i, 0)),
            pl.BlockSpec((RB, DF), lambda i: (i, 0)),
            pl.BlockSpec((DF, DF), lambda i: (0, 0)),
            pl.BlockSpec((DF, DF), lambda i: (0, 0)),
        ],
        out_specs=[
            pl.BlockSpec((RB, 1), lambda i: (i, 0)),
            pl.BlockSpec((RB, DF), lambda i: (i, 0)),
        ],
        out_shape=[
            jax.ShapeDtypeStruct((N, 1), jnp.float32),
            jax.ShapeDtypeStruct((N, DF), jnp.float32),
        ],
    )(dflat, x, W, Wm)


def _tc_layer(sacc, u, dis, b, bm, g, be, Wn, Wmn):
    def body(s_ref, u_ref, dis_ref, b_ref, bm_ref, g_ref, be_ref,
             w_ref, wm_ref, un_ref):
        dis_b = dis_ref[...]
        o = (s_ref[0] + s_ref[1] + u_ref[...]) * dis_b \
            + jnp.clip(bm_ref[...], 0.0, 1.0) * b_ref[...]
        a = jnp.maximum(o * _BN_SCALE * g_ref[...] + be_ref[...], 0.0)
        mw = jnp.clip(wm_ref[...], 0.0, 1.0) * w_ref[...]
        un_ref[...] = jnp.dot(a, mw, preferred_element_type=jnp.float32) * dis_b

    full = lambda i: (0, 0)
    return pl.pallas_call(
        body,
        grid=(N // RB,),
        in_specs=[
            pl.BlockSpec((2, RB, DF), lambda i: (0, i, 0)),
            pl.BlockSpec((RB, DF), lambda i: (i, 0)),
            pl.BlockSpec((RB, 1), lambda i: (i, 0)),
            pl.BlockSpec((1, DF), full),
            pl.BlockSpec((1, DF), full),
            pl.BlockSpec((1, DF), full),
            pl.BlockSpec((1, DF), full),
            pl.BlockSpec((DF, DF), full),
            pl.BlockSpec((DF, DF), full),
        ],
        out_specs=pl.BlockSpec((RB, DF), lambda i: (i, 0)),
        out_shape=jax.ShapeDtypeStruct((N, DF), jnp.float32),
    )(sacc, u, dis, b, bm, g, be, Wn, Wmn)


def _tc_final(sacc, u, dis, b, bm):
    def body(s_ref, u_ref, dis_ref, b_ref, bm_ref, out_ref):
        o = (s_ref[0] + s_ref[1] + u_ref[...]) * dis_ref[...] \
            + jnp.clip(bm_ref[...], 0.0, 1.0) * b_ref[...]
        m = jnp.max(o, axis=-1, keepdims=True)
        z = o - m
        lse = jnp.log(jnp.sum(jnp.exp(z), axis=-1, keepdims=True))
        out_ref[...] = z - lse

    full = lambda i: (0, 0)
    return pl.pallas_call(
        body,
        grid=(N // RB,),
        in_specs=[
            pl.BlockSpec((2, RB, DF), lambda i: (0, i, 0)),
            pl.BlockSpec((RB, DF), lambda i: (i, 0)),
            pl.BlockSpec((RB, 1), lambda i: (i, 0)),
            pl.BlockSpec((1, DF), full),
            pl.BlockSpec((1, DF), full),
        ],
        out_specs=pl.BlockSpec((RB, DF), lambda i: (i, 0)),
        out_shape=jax.ShapeDtypeStruct((N, DF), jnp.float32),
    )(sacc, u, dis, b, bm)


def kernel(x, edge_index, W1, Wm1, b1, bm1, g1, be1,
           W2, Wm2, b2, bm2, g2, be2, W3, Wm3, b3, bm3):
    src = edge_index[0]
    dst = edge_index[1]
    pad_i = jnp.arange(EP - E, dtype=jnp.int32)
    psrc = jnp.concatenate([src, pad_i % N])
    pdst = jnp.concatenate([dst, N + pad_i % (NP - N)])
    srcp = psrc.reshape(TOTAL_CHUNKS, NCHUNK)
    dstp = pdst.reshape(TOTAL_CHUNKS, NCHUNK)

    z128 = jnp.zeros((NP, DF), jnp.float32)

    b1r, bm1r = b1.reshape(1, DF), bm1.reshape(1, DF)
    b2r, bm2r = b2.reshape(1, DF), bm2.reshape(1, DF)
    b3r, bm3r = b3.reshape(1, DF), bm3.reshape(1, DF)
    g1r, be1r = g1.reshape(1, DF), be1.reshape(1, DF)
    g2r, be2r = g2.reshape(1, DF), be2.reshape(1, DF)

    mesh = plsc.VectorSubcoreMesh(core_axis_name="c", subcore_axis_name="s")

    dacc = _sc_degree(dstp, mesh)
    dflat = dacc.reshape(2, DR * 128, 1)
    dis, u1 = _tc_first_scale(dflat, x, W1, Wm1)
    s1 = _sc_scatter(u1, srcp, dstp, z128, mesh)
    u2 = _tc_layer(s1, u1, dis, b1r, bm1r, g1r, be1r, W2, Wm2)
    s2 = _sc_scatter(u2, srcp, dstp, z128, mesh)
    u3 = _tc_layer(s2, u2, dis, b2r, bm2r, g2r, be2r, W3, Wm3)
    s3 = _sc_scatter(u3, srcp, dstp, z128, mesh)
    return _tc_final(s3, u3, dis, b3r, bm3r)


# speedup vs baseline: 1.2652x; 1.0139x over previous
import functools

import jax
import jax.numpy as jnp
from jax import lax
from jax.experimental import pallas as pl
from jax.experimental.pallas import tpu as pltpu
from jax.experimental.pallas import tpu_sc as plsc

N = 10000
E = 320000
DF = 128
NP = 10240
NCHUNK = 128
NWORKERS = 32
TOTAL_CHUNKS = 2560
EP = TOTAL_CHUNKS * NCHUNK
KPT = TOTAL_CHUNKS // NWORKERS
GC = 40
GROUPS = KPT // GC
RPT = NP // 16
RB = 2000
_BN_SCALE = 0.9999950000374996


DR = 128


def _sc_degree(dstp, mesh):
    @functools.partial(
        pl.kernel,
        out_type=jax.ShapeDtypeStruct((2, DR, 128), jnp.float32),
        mesh=mesh,
        compiler_params=pltpu.CompilerParams(needs_layout_passes=False),
        scratch_types=[
            pltpu.VMEM((KPT, NCHUNK), jnp.int32),
            pltpu.VMEM((DR, 128), jnp.float32),
            pltpu.VMEM((DR // 16, 128), jnp.float32),
            pltpu.VMEM((1, 128), jnp.int32),
            pltpu.VMEM_SHARED((DR, 128), jnp.float32),
        ],
    )
    def k(dstp_hbm, out_hbm, dbuf, hist, zbuf, idxrow, dacc):
        c = lax.axis_index("c")
        s = lax.axis_index("s")
        wid = s * 2 + c
        zrows = DR // 16
        lane16 = jax.lax.iota(jnp.int32, 16)

        pltpu.sync_copy(dstp_hbm.at[pl.ds(wid * KPT, KPT)], dbuf)

        def fill_zero(i, carry):
            r = i // 8
            k16 = i % 8
            hist[r, pl.ds(k16 * 16, 16)] = jnp.zeros((16,), jnp.float32)
            return carry

        lax.fori_loop(0, DR * 8, fill_zero, 0)

        def fill_zbuf(i, carry):
            zbuf[i // 8, pl.ds((i % 8) * 16, 16)] = jnp.zeros((16,), jnp.float32)
            return carry

        lax.fori_loop(0, zrows * 8, fill_zbuf, 0)

        for k16 in range(8):
            idxrow[0, pl.ds(k16 * 16, 16)] = lane16 + (16 * k16)

        pltpu.sync_copy(zbuf, dacc.at[pl.ds(s * zrows, zrows)])

        def count(i, carry):
            v = dbuf[i // 8, pl.ds((i % 8) * 16, 16)]
            cnt, is_last = plsc.scan_count(v)
            plsc.addupdate_scatter(
                hist,
                [lax.shift_right_logical(v, 7), lax.bitwise_and(v, 127)],
                cnt.astype(jnp.float32),
                mask=is_last,
            )
            return carry

        lax.fori_loop(0, KPT * 8, count, 0)
        plsc.subcore_barrier()
        pltpu.sync_copy(hist, dacc.at[idxrow.at[0]], add=True)
        plsc.subcore_barrier()
        pltpu.sync_copy(dacc.at[pl.ds(s * zrows, zrows)],
                        out_hbm.at[c, pl.ds(s * zrows, zrows)])

    return k(dstp)


def _sc_scatter(u, srcp, dstp, z128, mesh):
    @functools.partial(
        pl.kernel,
        out_type=jax.ShapeDtypeStruct((2, NP, DF), jnp.float32),
        mesh=mesh,
        scratch_types=[
            pltpu.VMEM((GC, NCHUNK), jnp.int32),
            pltpu.VMEM((GC, NCHUNK), jnp.int32),
            pltpu.VMEM((NCHUNK, DF), jnp.float32),
            pltpu.VMEM((NCHUNK, DF), jnp.float32),
            pltpu.VMEM_SHARED((NP, DF), jnp.float32),
            pltpu.SemaphoreType.DMA,
            pltpu.SemaphoreType.DMA,
            pltpu.SemaphoreType.DMA,
        ],
    )
    def k(u_hbm, srcp_hbm, dstp_hbm, z_hbm, out_hbm,
          sbuf, dbuf, rows0, rows1, acc, sem0, sem1, semz):
        c = lax.axis_index("c")
        s = lax.axis_index("s")
        wid = s * 2 + c

        def load_idx(gi):
            base = wid * KPT + gi * GC
            pltpu.sync_copy(srcp_hbm.at[pl.ds(base, GC)], sbuf)
            pltpu.sync_copy(dstp_hbm.at[pl.ds(base, GC)], dbuf)

        def gather(j, rows, sem):
            pltpu.async_copy(u_hbm.at[sbuf.at[j]], rows, sem)

        def gather_wait(j, rows, sem):
            pltpu.make_async_copy(u_hbm.at[sbuf.at[j]], rows, sem).wait()

        zc = pltpu.async_copy(z_hbm.at[pl.ds(s * RPT, RPT)],
                              acc.at[pl.ds(s * RPT, RPT)], semz)
        load_idx(0)
        gather(0, rows0, sem0)
        gather(1, rows1, sem1)
        zc.wait()
        plsc.subcore_barrier()

        def body(i, carry):
            j = 2 * i
            gather_wait(j, rows0, sem0)
            pltpu.sync_copy(rows0, acc.at[dbuf.at[j]], add=True)

            @pl.when(j + 2 < GC)
            def _():
                gather(j + 2, rows0, sem0)

            gather_wait(j + 1, rows1, sem1)
            pltpu.sync_copy(rows1, acc.at[dbuf.at[j + 1]], add=True)

            @pl.when(j + 3 < GC)
            def _():
                gather(j + 3, rows1, sem1)

            return carry

        lax.fori_loop(0, GC // 2, body, 0)
        for gi in range(1, GROUPS):
            load_idx(gi)
            gather(0, rows0, sem0)
            gather(1, rows1, sem1)
            lax.fori_loop(0, GC // 2, body, 0)
        plsc.subcore_barrier()
        pltpu.sync_copy(acc.at[pl.ds(s * RPT, RPT)],
                        out_hbm.at[c, pl.ds(s * RPT, RPT)])

    return k(u, srcp, dstp, z128)


def _tc_first_scale(dflat, x, W, Wm):
    def body(d_ref, x_ref, w_ref, wm_ref, dis_ref, u_ref):
        cnt = d_ref[0] + d_ref[1]
        dis = lax.rsqrt(1.0 + cnt)
        dis_ref[...] = dis
        mw = jnp.clip(wm_ref[...], 0.0, 1.0) * w_ref[...]
        u_ref[...] = jnp.dot(x_ref[...], mw,
                             preferred_element_type=jnp.float32) * dis

    return pl.pallas_call(
        body,
        grid=(N // RB,),
        in_specs=[
            pl.BlockSpec((2, RB, 1), lambda i: (0, i, 0)),
            pl.BlockSpec((RB, DF), lambda i: (i, 0)),
            pl.BlockSpec((DF, DF), lambda i: (0, 0)),
            pl.BlockSpec((DF, DF), lambda i: (0, 0)),
        ],
        out_specs=[
            pl.BlockSpec((RB, 1), lambda i: (i, 0)),
            pl.BlockSpec((RB, DF), lambda i: (i, 0)),
        ],
        out_shape=[
            jax.ShapeDtypeStruct((N, 1), jnp.float32),
            jax.ShapeDtypeStruct((N, DF), jnp.float32),
        ],
    )(dflat, x, W, Wm)


def _tc_layer(sacc, u, dis, b, bm, g, be, Wn, Wmn):
    def body(s_ref, u_ref, dis_ref, b_ref, bm_ref, g_ref, be_ref,
             w_ref, wm_ref, un_ref):
        dis_b = dis_ref[...]
        o = (s_ref[0] + s_ref[1] + u_ref[...]) * dis_b \
            + jnp.clip(bm_ref[...], 0.0, 1.0) * b_ref[...]
        a = jnp.maximum(o * _BN_SCALE * g_ref[...] + be_ref[...], 0.0)
        mw = jnp.clip(wm_ref[...], 0.0, 1.0) * w_ref[...]
        un_ref[...] = jnp.dot(a, mw, preferred_element_type=jnp.float32) * dis_b

    full = lambda i: (0, 0)
    return pl.pallas_call(
        body,
        grid=(N // RB,),
        in_specs=[
            pl.BlockSpec((2, RB, DF), lambda i: (0, i, 0)),
            pl.BlockSpec((RB, DF), lambda i: (i, 0)),
            pl.BlockSpec((RB, 1), lambda i: (i, 0)),
            pl.BlockSpec((1, DF), full),
            pl.BlockSpec((1, DF), full),
            pl.BlockSpec((1, DF), full),
            pl.BlockSpec((1, DF), full),
            pl.BlockSpec((DF, DF), full),
            pl.BlockSpec((DF, DF), full),
        ],
        out_specs=pl.BlockSpec((RB, DF), lambda i: (i, 0)),
        out_shape=jax.ShapeDtypeStruct((N, DF), jnp.float32),
    )(sacc, u, dis, b, bm, g, be, Wn, Wmn)


def _tc_final(sacc, u, dis, b, bm):
    def body(s_ref, u_ref, dis_ref, b_ref, bm_ref, out_ref):
        o = (s_ref[0] + s_ref[1] + u_ref[...]) * dis_ref[...] \
            + jnp.clip(bm_ref[...], 0.0, 1.0) * b_ref[...]
        m = jnp.max(o, axis=-1, keepdims=True)
        z = o - m
        lse = jnp.log(jnp.sum(jnp.exp(z), axis=-1, keepdims=True))
        out_ref[...] = z - lse

    full = lambda i: (0, 0)
    return pl.pallas_call(
        body,
        grid=(N // RB,),
        in_specs=[
            pl.BlockSpec((2, RB, DF), lambda i: (0, i, 0)),
            pl.BlockSpec((RB, DF), lambda i: (i, 0)),
            pl.BlockSpec((RB, 1), lambda i: (i, 0)),
            pl.BlockSpec((1, DF), full),
            pl.BlockSpec((1, DF), full),
        ],
        out_specs=pl.BlockSpec((RB, DF), lambda i: (i, 0)),
        out_shape=jax.ShapeDtypeStruct((N, DF), jnp.float32),
    )(sacc, u, dis, b, bm)


def kernel(x, edge_index, W1, Wm1, b1, bm1, g1, be1,
           W2, Wm2, b2, bm2, g2, be2, W3, Wm3, b3, bm3):
    src = edge_index[0]
    dst = edge_index[1]
    pad_i = jnp.arange(EP - E, dtype=jnp.int32)
    psrc = jnp.concatenate([src, pad_i % N])
    pdst = jnp.concatenate([dst, N + pad_i % (NP - N)])
    srcp = psrc.reshape(TOTAL_CHUNKS, NCHUNK)
    dstp = pdst.reshape(TOTAL_CHUNKS, NCHUNK)

    z128 = jnp.zeros((NP, DF), jnp.float32)

    b1r, bm1r = b1.reshape(1, DF), bm1.reshape(1, DF)
    b2r, bm2r = b2.reshape(1, DF), bm2.reshape(1, DF)
    b3r, bm3r = b3.reshape(1, DF), bm3.reshape(1, DF)
    g1r, be1r = g1.reshape(1, DF), be1.reshape(1, DF)
    g2r, be2r = g2.reshape(1, DF), be2.reshape(1, DF)

    mesh = plsc.VectorSubcoreMesh(core_axis_name="c", subcore_axis_name="s")

    dacc = _sc_degree(dstp, mesh)
    dflat = dacc.reshape(2, DR * 128, 1)
    dis, u1 = _tc_first_scale(dflat, x, W1, Wm1)
    s1 = _sc_scatter(u1, srcp, dstp, z128, mesh)
    u2 = _tc_layer(s1, u1, dis, b1r, bm1r, g1r, be1r, W2, Wm2)
    s2 = _sc_scatter(u2, srcp, dstp, z128, mesh)
    u3 = _tc_layer(s2, u2, dis, b2r, bm2r, g2r, be2r, W3, Wm3)
    s3 = _sc_scatter(u3, srcp, dstp, z128, mesh)
    return _tc_final(s3, u3, dis, b3r, bm3r)
